# Initial kernel scaffold; baseline (speedup 1.0000x reference)
import jax
import jax.numpy as jnp
from jax.experimental import pallas as pl


def kernel(encoded_tokens, position_table):
    raise NotImplementedError("write your pallas kernel here")


# speedup vs baseline: 1.8050x; 1.8050x over previous
import jax
import jax.numpy as jnp
from jax.experimental import pallas as pl


_TOKEN_BLOCK = 512


def _add_kernel(tok_ref, tab_ref, out_ref):
    out_ref[...] = tok_ref[...] + tab_ref[...][None, :, :]


def kernel(encoded_tokens, position_table):
    batch, num_tokens, embed_dim = encoded_tokens.shape
    tb = _TOKEN_BLOCK
    grid = (num_tokens // tb,)
    return pl.pallas_call(
        _add_kernel,
        grid=grid,
        in_specs=[
            pl.BlockSpec((batch, tb, embed_dim), lambda i: (0, i, 0)),
            pl.BlockSpec((tb, embed_dim), lambda i: (i, 0)),
        ],
        out_specs=pl.BlockSpec((batch, tb, embed_dim), lambda i: (0, i, 0)),
        out_shape=jax.ShapeDtypeStruct(
            (batch, num_tokens, embed_dim), encoded_tokens.dtype
        ),
    )(encoded_tokens, position_table)
